# baseline (speedup 1.0000x reference)
import functools

import jax
import jax.numpy as jnp
from jax import lax
from jax.experimental import pallas as pl
from jax.experimental.pallas import tpu as pltpu
from jax.experimental.pallas import tpu_sc as plsc

_LANES = 16
_NC = 2
_NS = 16
_NW = _NC * _NS
_CHUNK = 32


def _sc_body(rows, d, x_hbm, ids_hbm, emb_hbm, out_hbm,
             idx_v, emb_v, buf_v, sem):
    wid = lax.axis_index("s") * _NC + lax.axis_index("c")
    rows_per_w = rows // _NW
    n_chunks = rows_per_w // _CHUNK
    groups = d // _LANES
    base = wid * rows_per_w

    pltpu.sync_copy(ids_hbm, idx_v)
    pltpu.async_copy(emb_hbm.at[idx_v], emb_v, sem).wait()

    def chunk_body(g, carry):
        r0 = base + g * _CHUNK
        pltpu.sync_copy(x_hbm.at[pl.ds(r0, _CHUNK)], buf_v)

        def row_body(r, carry2):
            def lane_body(j, carry3):
                sl = pl.ds(j * _LANES, _LANES)
                buf_v[r, sl] = buf_v[r, sl] + emb_v[0, sl]
                return carry3
            return lax.fori_loop(0, groups, lane_body, carry2, unroll=8)

        lax.fori_loop(0, _CHUNK, row_body, carry)
        pltpu.sync_copy(buf_v, out_hbm.at[pl.ds(r0, _CHUNK)])
        return carry

    lax.fori_loop(0, n_chunks, chunk_body, 0)


def kernel(lang, emb_weight, dataset_id):
    b, s, d = lang.shape
    rows = b * s
    x = lang.reshape(rows, d)
    ids = jnp.full((_LANES,), dataset_id, jnp.int32)

    mesh = plsc.VectorSubcoreMesh(core_axis_name="c", subcore_axis_name="s")
    k = pl.kernel(
        functools.partial(_sc_body, rows, d),
        mesh=mesh,
        out_type=jax.ShapeDtypeStruct((rows, d), lang.dtype),
        scratch_types=[
            pltpu.VMEM((_LANES,), jnp.int32),
            pltpu.VMEM((_LANES, d), jnp.float32),
            pltpu.VMEM((_CHUNK, d), jnp.float32),
            pltpu.SemaphoreType.DMA,
        ],
    )
    out = k(x, ids, emb_weight)
    return out.reshape(b, s, d)


# speedup vs baseline: 1.5696x; 1.5696x over previous
import functools

import jax
import jax.numpy as jnp
from jax import lax
from jax.experimental import pallas as pl
from jax.experimental.pallas import tpu as pltpu
from jax.experimental.pallas import tpu_sc as plsc

_LANES = 16
_NC = 2
_NS = 16
_NW = _NC * _NS
_CHUNK = 32
_SC_ROWS = 4096
_BLOCK = 2048


def _sc_body(rows, d, x_hbm, ids_hbm, emb_hbm, out_hbm,
             idx_v, emb_v, buf_v, sem):
    wid = lax.axis_index("s") * _NC + lax.axis_index("c")
    rows_per_w = rows // _NW
    n_chunks = rows_per_w // _CHUNK
    groups = d // _LANES
    base = wid * rows_per_w

    pltpu.sync_copy(ids_hbm, idx_v)
    pltpu.async_copy(emb_hbm.at[idx_v], emb_v, sem).wait()

    def chunk_body(g, carry):
        r0 = base + g * _CHUNK
        pltpu.sync_copy(x_hbm.at[pl.ds(r0, _CHUNK)], buf_v)

        def row_body(r, carry2):
            def lane_body(j, carry3):
                sl = pl.ds(j * _LANES, _LANES)
                buf_v[r, sl] = buf_v[r, sl] + emb_v[0, sl]
                return carry3
            return lax.fori_loop(0, groups, lane_body, carry2, unroll=8)

        lax.fori_loop(0, _CHUNK, row_body, carry)
        pltpu.sync_copy(buf_v, out_hbm.at[pl.ds(r0, _CHUNK)])
        return carry

    lax.fori_loop(0, n_chunks, chunk_body, 0)


def _tc_body(ids_ref, x_ref, emb_ref, o_ref):
    row = emb_ref[ids_ref[0], :]
    o_ref[...] = x_ref[...] + row[None, :]


def _tc_add(x, emb_weight, ids_vec):
    rows, d = x.shape
    n_vocab = emb_weight.shape[0]
    grid_spec = pltpu.PrefetchScalarGridSpec(
        num_scalar_prefetch=1,
        grid=(rows // _BLOCK,),
        in_specs=[
            pl.BlockSpec((_BLOCK, d), lambda i, ids: (i, 0)),
            pl.BlockSpec((n_vocab, d), lambda i, ids: (0, 0)),
        ],
        out_specs=pl.BlockSpec((_BLOCK, d), lambda i, ids: (i, 0)),
    )
    return pl.pallas_call(
        _tc_body,
        grid_spec=grid_spec,
        out_shape=jax.ShapeDtypeStruct((rows, d), x.dtype),
        compiler_params=pltpu.CompilerParams(
            dimension_semantics=("parallel",),
        ),
    )(ids_vec, x, emb_weight)


def _sc_add(x, emb_weight, ids16):
    rows, d = x.shape
    mesh = plsc.VectorSubcoreMesh(core_axis_name="c", subcore_axis_name="s")
    k = pl.kernel(
        functools.partial(_sc_body, rows, d),
        mesh=mesh,
        out_type=jax.ShapeDtypeStruct((rows, d), x.dtype),
        scratch_types=[
            pltpu.VMEM((_LANES,), jnp.int32),
            pltpu.VMEM((_LANES, d), jnp.float32),
            pltpu.VMEM((_CHUNK, d), jnp.float32),
            pltpu.SemaphoreType.DMA,
        ],
    )
    return k(x, ids16, emb_weight)


def kernel(lang, emb_weight, dataset_id):
    b, s, d = lang.shape
    rows = b * s
    x = lang.reshape(rows, d)
    did = jnp.asarray(dataset_id, jnp.int32)
    ids16 = jnp.full((_LANES,), did, jnp.int32)

    out_sc = _sc_add(x[:_SC_ROWS], emb_weight, ids16)
    out_tc = _tc_add(x[_SC_ROWS:], emb_weight, did.reshape(1))
    out = jnp.concatenate([out_sc, out_tc], axis=0)
    return out.reshape(b, s, d)


# speedup vs baseline: 5.4753x; 3.4884x over previous
import jax
import jax.numpy as jnp
from jax.experimental import pallas as pl
from jax.experimental.pallas import tpu as pltpu

_BLOCK = 2048


def _body(ids_ref, x_ref, emb_ref, o_ref):
    row = emb_ref[ids_ref[0], :]
    o_ref[...] = x_ref[...] + row[None, :]


def kernel(lang, emb_weight, dataset_id):
    b, s, d = lang.shape
    n_vocab = emb_weight.shape[0]
    rows = b * s
    x = lang.reshape(rows, d)
    ids = jnp.asarray(dataset_id, jnp.int32).reshape(1)

    grid_spec = pltpu.PrefetchScalarGridSpec(
        num_scalar_prefetch=1,
        grid=(rows // _BLOCK,),
        in_specs=[
            pl.BlockSpec((_BLOCK, d), lambda i, ids: (i, 0)),
            pl.BlockSpec((n_vocab, d), lambda i, ids: (0, 0)),
        ],
        out_specs=pl.BlockSpec((_BLOCK, d), lambda i, ids: (i, 0)),
    )
    out = pl.pallas_call(
        _body,
        grid_spec=grid_spec,
        out_shape=jax.ShapeDtypeStruct((rows, d), lang.dtype),
        compiler_params=pltpu.CompilerParams(
            dimension_semantics=("parallel",),
        ),
    )(ids, x, emb_weight)
    return out.reshape(b, s, d)


# speedup vs baseline: 5.5379x; 1.0114x over previous
import jax
import jax.numpy as jnp
from jax.experimental import pallas as pl
from jax.experimental.pallas import tpu as pltpu

_BLOCK = 3072


def _body(ids_ref, x_ref, emb_ref, o_ref):
    row = emb_ref[ids_ref[0], :]
    o_ref[...] = x_ref[...] + row[None, :]


def kernel(lang, emb_weight, dataset_id):
    b, s, d = lang.shape
    n_vocab = emb_weight.shape[0]
    rows = b * s
    x = lang.reshape(rows, d)
    ids = jnp.asarray(dataset_id, jnp.int32).reshape(1)

    grid_spec = pltpu.PrefetchScalarGridSpec(
        num_scalar_prefetch=1,
        grid=(pl.cdiv(rows, _BLOCK),),
        in_specs=[
            pl.BlockSpec((_BLOCK, d), lambda i, ids: (i, 0)),
            pl.BlockSpec((n_vocab, d), lambda i, ids: (0, 0)),
        ],
        out_specs=pl.BlockSpec((_BLOCK, d), lambda i, ids: (i, 0)),
    )
    out = pl.pallas_call(
        _body,
        grid_spec=grid_spec,
        out_shape=jax.ShapeDtypeStruct((rows, d), lang.dtype),
        compiler_params=pltpu.CompilerParams(
            dimension_semantics=("parallel",),
        ),
    )(ids, x, emb_weight)
    return out.reshape(b, s, d)
